# baseline (speedup 1.0000x reference)
import functools

import jax
import jax.numpy as jnp
from jax import lax
from jax.experimental import pallas as pl
from jax.experimental.pallas import tpu as pltpu
from jax.experimental.pallas import tpu_sc as plsc

N_NODES = 10000
D = 128
E1 = 320000
E2 = 640000

NC = 2
NS = 16
CH = 125
KB = 8
NB = 2

ROWS1 = E1 // (NS * CH)
ROWS2 = E2 // (NS * CH)
HEAD2 = 248
TAIL2 = ROWS2 - HEAD2
TBASE = HEAD2 * NS
ACC_R = 10240
ZR = ACC_R // NS
LASTR = N_NODES - (NS - 1) * ZR
IW = 2 * KB


def _body(x_hbm, src1, dst1, src2, dst2, zblk,
          x1p, p2a, p2b,
          acc, idx_s, idx_d, r0, r1, g0, g1):
    c = lax.axis_index("c")
    s = lax.axis_index("s")
    rows = (r0, r1)
    sems = (g0, g1)

    def zero_acc():
        def zero_blk(k, carry):
            pltpu.sync_copy(zblk, acc.at[pl.ds(s * ZR + k * 128, 128)])
            return carry
        lax.fori_loop(0, ZR // 128, zero_blk, 0)

    zero_acc()
    plsc.subcore_barrier()

    def run_list(src_hbm, dst_hbm, nrows, hbase):
        row0 = hbase + s * nrows

        pltpu.sync_copy(src_hbm.at[pl.ds(row0, KB)], idx_s.at[pl.ds(0, KB)])
        pltpu.sync_copy(dst_hbm.at[pl.ds(row0, KB)], idx_d.at[pl.ds(0, KB)])
        pltpu.sync_copy(src_hbm.at[pl.ds(row0 + KB, KB)],
                        idx_s.at[pl.ds(KB, KB)])
        pltpu.sync_copy(dst_hbm.at[pl.ds(row0 + KB, KB)],
                        idx_d.at[pl.ds(KB, KB)])

        for b in range(NB):
            pltpu.async_copy(x_hbm.at[idx_s.at[b]], rows[b], sems[b])

        def grp(g, carry):
            for b in range(NB):
                j = g * NB + b
                w = lax.rem(j, IW)
                pltpu.make_async_copy(
                    x_hbm.at[idx_s.at[w]], rows[b], sems[b]).wait()
                pltpu.sync_copy(rows[b], acc.at[idx_d.at[w]], add=True)

                t = j + NB
                tw = lax.rem(t, IW)

                @pl.when(jnp.logical_and(lax.rem(t, KB) == 0, t < nrows))
                def _():
                    hb = pl.multiple_of(row0 + t, 8)
                    vb = pl.multiple_of(tw, 8)
                    pltpu.sync_copy(src_hbm.at[pl.ds(hb, KB)],
                                    idx_s.at[pl.ds(vb, KB)])
                    pltpu.sync_copy(dst_hbm.at[pl.ds(hb, KB)],
                                    idx_d.at[pl.ds(vb, KB)])

                @pl.when(t < nrows)
                def _():
                    pltpu.async_copy(x_hbm.at[idx_s.at[tw]], rows[b], sems[b])
            return carry
        lax.fori_loop(0, nrows // NB, grp, 0)

    @pl.when(c == 0)
    def _():
        run_list(src2, dst2, TAIL2, TBASE)
        plsc.subcore_barrier()
        pltpu.sync_copy(acc.at[pl.ds(s * ZR, ZR)], p2a.at[pl.ds(s * ZR, ZR)])
        zero_acc()
        plsc.subcore_barrier()
        run_list(src1, dst1, ROWS1, 0)
        plsc.subcore_barrier()

        @pl.when(s < NS - 1)
        def _():
            pltpu.sync_copy(acc.at[pl.ds(s * ZR, ZR)],
                            x1p.at[pl.ds(s * ZR, ZR)])

        @pl.when(s == NS - 1)
        def _():
            pltpu.sync_copy(acc.at[pl.ds((NS - 1) * ZR, LASTR)],
                            x1p.at[pl.ds((NS - 1) * ZR, LASTR)])

    @pl.when(c == 1)
    def _():
        run_list(src2, dst2, HEAD2, 0)
        plsc.subcore_barrier()
        pltpu.sync_copy(acc.at[pl.ds(s * ZR, ZR)], p2b.at[pl.ds(s * ZR, ZR)])


def _merge_body(x1_ref, a_ref, b_ref, o_ref):
    o_ref[:, :D] = x1_ref[...]
    o_ref[:, D:] = a_ref[...] + b_ref[...]


@jax.jit
def _h2gcn(x, src1, dst1, src2, dst2):
    zblk = jnp.zeros((128, D), jnp.float32)
    mesh = plsc.VectorSubcoreMesh(core_axis_name="c", subcore_axis_name="s")
    k = functools.partial(
        pl.kernel,
        mesh=mesh,
        out_type=[
            jax.ShapeDtypeStruct((N_NODES, D), jnp.float32),
            jax.ShapeDtypeStruct((ACC_R, D), jnp.float32),
            jax.ShapeDtypeStruct((ACC_R, D), jnp.float32),
        ],
        scratch_types=[
            pltpu.VMEM_SHARED((ACC_R, D), jnp.float32),
            pltpu.VMEM((IW, CH), jnp.int32),
            pltpu.VMEM((IW, CH), jnp.int32),
            pltpu.VMEM((CH, D), jnp.float32),
            pltpu.VMEM((CH, D), jnp.float32),
            pltpu.SemaphoreType.DMA,
            pltpu.SemaphoreType.DMA,
        ],
    )(_body)
    x1p, p2a, p2b = k(x, src1, dst1, src2, dst2, zblk)

    blk = 1000
    return pl.pallas_call(
        _merge_body,
        grid=(N_NODES // blk,),
        in_specs=[
            pl.BlockSpec((blk, D), lambda i: (i, 0)),
            pl.BlockSpec((blk, D), lambda i: (i, 0)),
            pl.BlockSpec((blk, D), lambda i: (i, 0)),
        ],
        out_specs=pl.BlockSpec((blk, 2 * D), lambda i: (i, 0)),
        out_shape=jax.ShapeDtypeStruct((N_NODES, 2 * D), jnp.float32),
    )(x1p, p2a, p2b)


def kernel(x, edge_index, edge_index2):
    src1 = edge_index[0].astype(jnp.int32).reshape(E1 // CH, CH)
    dst1 = edge_index[1].astype(jnp.int32).reshape(E1 // CH, CH)
    src2 = edge_index2[0].astype(jnp.int32).reshape(E2 // CH, CH)
    dst2 = edge_index2[1].astype(jnp.int32).reshape(E2 // CH, CH)
    return _h2gcn(x, src1, dst1, src2, dst2)


# speedup vs baseline: 1.0172x; 1.0172x over previous
import functools

import jax
import jax.numpy as jnp
from jax import lax
from jax.experimental import pallas as pl
from jax.experimental.pallas import tpu as pltpu
from jax.experimental.pallas import tpu_sc as plsc

N_NODES = 10000
D = 128
E1 = 320000
E2 = 640000

NC = 2
NS = 16
CH = 125
KB = 16
NB = 2

ROWS1 = E1 // (NS * CH)
ROWS2 = E2 // (NS * CH)
HEAD2 = 240
TAIL2 = ROWS2 - HEAD2
TBASE = HEAD2 * NS
ACC_R = 10240
ZR = ACC_R // NS
LASTR = N_NODES - (NS - 1) * ZR
IW = 2 * KB


def _body(x_hbm, src1, dst1, src2, dst2, zblk,
          x1p, p2a, p2b,
          acc, idx_s, idx_d, r0, r1, g0, g1):
    c = lax.axis_index("c")
    s = lax.axis_index("s")
    rows = (r0, r1)
    sems = (g0, g1)

    def zero_acc():
        def zero_blk(k, carry):
            pltpu.sync_copy(zblk, acc.at[pl.ds(s * ZR + k * 128, 128)])
            return carry
        lax.fori_loop(0, ZR // 128, zero_blk, 0)

    zero_acc()
    plsc.subcore_barrier()

    def run_list(src_hbm, dst_hbm, nrows, hbase):
        row0 = hbase + s * nrows

        pltpu.sync_copy(src_hbm.at[pl.ds(row0, KB)], idx_s.at[pl.ds(0, KB)])
        pltpu.sync_copy(dst_hbm.at[pl.ds(row0, KB)], idx_d.at[pl.ds(0, KB)])
        pltpu.sync_copy(src_hbm.at[pl.ds(row0 + KB, KB)],
                        idx_s.at[pl.ds(KB, KB)])
        pltpu.sync_copy(dst_hbm.at[pl.ds(row0 + KB, KB)],
                        idx_d.at[pl.ds(KB, KB)])

        for b in range(NB):
            pltpu.async_copy(x_hbm.at[idx_s.at[b]], rows[b], sems[b])

        def grp(g, carry):
            for b in range(NB):
                j = g * NB + b
                w = lax.rem(j, IW)
                pltpu.make_async_copy(
                    x_hbm.at[idx_s.at[w]], rows[b], sems[b]).wait()
                pltpu.sync_copy(rows[b], acc.at[idx_d.at[w]], add=True)

                t = j + NB
                tw = lax.rem(t, IW)

                @pl.when(jnp.logical_and(lax.rem(t, KB) == 0, t < nrows))
                def _():
                    hb = pl.multiple_of(row0 + t, 8)
                    vb = pl.multiple_of(tw, 8)
                    pltpu.sync_copy(src_hbm.at[pl.ds(hb, KB)],
                                    idx_s.at[pl.ds(vb, KB)])
                    pltpu.sync_copy(dst_hbm.at[pl.ds(hb, KB)],
                                    idx_d.at[pl.ds(vb, KB)])

                @pl.when(t < nrows)
                def _():
                    pltpu.async_copy(x_hbm.at[idx_s.at[tw]], rows[b], sems[b])
            return carry
        lax.fori_loop(0, nrows // NB, grp, 0)

    @pl.when(c == 0)
    def _():
        run_list(src2, dst2, TAIL2, TBASE)
        plsc.subcore_barrier()
        pltpu.sync_copy(acc.at[pl.ds(s * ZR, ZR)], p2a.at[pl.ds(s * ZR, ZR)])
        zero_acc()
        plsc.subcore_barrier()
        run_list(src1, dst1, ROWS1, 0)
        plsc.subcore_barrier()

        @pl.when(s < NS - 1)
        def _():
            pltpu.sync_copy(acc.at[pl.ds(s * ZR, ZR)],
                            x1p.at[pl.ds(s * ZR, ZR)])

        @pl.when(s == NS - 1)
        def _():
            pltpu.sync_copy(acc.at[pl.ds((NS - 1) * ZR, LASTR)],
                            x1p.at[pl.ds((NS - 1) * ZR, LASTR)])

    @pl.when(c == 1)
    def _():
        run_list(src2, dst2, HEAD2, 0)
        plsc.subcore_barrier()
        pltpu.sync_copy(acc.at[pl.ds(s * ZR, ZR)], p2b.at[pl.ds(s * ZR, ZR)])


def _merge_body(x1_ref, a_ref, b_ref, o_ref):
    o_ref[:, :D] = x1_ref[...]
    o_ref[:, D:] = a_ref[...] + b_ref[...]


@jax.jit
def _h2gcn(x, src1, dst1, src2, dst2):
    zblk = jnp.zeros((128, D), jnp.float32)
    mesh = plsc.VectorSubcoreMesh(core_axis_name="c", subcore_axis_name="s")
    k = functools.partial(
        pl.kernel,
        mesh=mesh,
        out_type=[
            jax.ShapeDtypeStruct((N_NODES, D), jnp.float32),
            jax.ShapeDtypeStruct((ACC_R, D), jnp.float32),
            jax.ShapeDtypeStruct((ACC_R, D), jnp.float32),
        ],
        scratch_types=[
            pltpu.VMEM_SHARED((ACC_R, D), jnp.float32),
            pltpu.VMEM((IW, CH), jnp.int32),
            pltpu.VMEM((IW, CH), jnp.int32),
            pltpu.VMEM((CH, D), jnp.float32),
            pltpu.VMEM((CH, D), jnp.float32),
            pltpu.SemaphoreType.DMA,
            pltpu.SemaphoreType.DMA,
        ],
    )(_body)
    x1p, p2a, p2b = k(x, src1, dst1, src2, dst2, zblk)

    blk = 1000
    return pl.pallas_call(
        _merge_body,
        grid=(N_NODES // blk,),
        in_specs=[
            pl.BlockSpec((blk, D), lambda i: (i, 0)),
            pl.BlockSpec((blk, D), lambda i: (i, 0)),
            pl.BlockSpec((blk, D), lambda i: (i, 0)),
        ],
        out_specs=pl.BlockSpec((blk, 2 * D), lambda i: (i, 0)),
        out_shape=jax.ShapeDtypeStruct((N_NODES, 2 * D), jnp.float32),
    )(x1p, p2a, p2b)


def kernel(x, edge_index, edge_index2):
    src1 = edge_index[0].astype(jnp.int32).reshape(E1 // CH, CH)
    dst1 = edge_index[1].astype(jnp.int32).reshape(E1 // CH, CH)
    src2 = edge_index2[0].astype(jnp.int32).reshape(E2 // CH, CH)
    dst2 = edge_index2[1].astype(jnp.int32).reshape(E2 // CH, CH)
    return _h2gcn(x, src1, dst1, src2, dst2)


# speedup vs baseline: 1.0478x; 1.0300x over previous
import functools

import jax
import jax.numpy as jnp
from jax import lax
from jax.experimental import pallas as pl
from jax.experimental.pallas import tpu as pltpu
from jax.experimental.pallas import tpu_sc as plsc

N_NODES = 10000
D = 128
E1 = 320000
E2 = 640000

NC = 2
NS = 16
CH = 125
KB = 16
NB = 2

ROWS1 = E1 // (NS * CH)
ROWS2 = E2 // (NS * CH)
HEAD2 = 240
TAIL2 = ROWS2 - HEAD2
TBASE = HEAD2 * NS
ACC_R = 10240
ZR = ACC_R // NS
LASTR = N_NODES - (NS - 1) * ZR
IW = 4 * KB


def _body(x_hbm, src1, dst1, src2, dst2, zblk,
          x1p, p2a, p2b,
          acc, idx_s, idx_d, r0, r1, g0, g1, rsem):
    c = lax.axis_index("c")
    s = lax.axis_index("s")
    rows = (r0, r1)
    sems = (g0, g1)

    def zero_acc():
        def zero_blk(k, carry):
            pltpu.sync_copy(zblk, acc.at[pl.ds(s * ZR + k * 128, 128)])
            return carry
        lax.fori_loop(0, ZR // 128, zero_blk, 0)

    zero_acc()
    plsc.subcore_barrier()

    def run_list(src_hbm, dst_hbm, nrows, hbase):
        row0 = hbase + s * nrows

        def refill_descs(t):
            hb = pl.multiple_of(row0 + t, 8)
            vb = pl.multiple_of(lax.rem(t, IW), 8)
            return ((src_hbm.at[pl.ds(hb, KB)], idx_s.at[pl.ds(vb, KB)]),
                    (dst_hbm.at[pl.ds(hb, KB)], idx_d.at[pl.ds(vb, KB)]))

        for blk0 in (0, KB):
            for sd, dd in refill_descs(blk0):
                pltpu.sync_copy(sd, dd)

        for b in range(NB):
            pltpu.async_copy(x_hbm.at[idx_s.at[b]], rows[b], sems[b])

        def grp(g, carry):
            for b in range(NB):
                j = g * NB + b
                w = lax.rem(j, IW)
                pltpu.make_async_copy(
                    x_hbm.at[idx_s.at[w]], rows[b], sems[b]).wait()
                pltpu.sync_copy(rows[b], acc.at[idx_d.at[w]], add=True)

                t = j + NB
                tw = lax.rem(t, IW)

                @pl.when(jnp.logical_and(lax.rem(t, KB) == 0, t < nrows))
                def _():
                    @pl.when(t >= 2 * KB)
                    def _():
                        for sd, dd in refill_descs(t):
                            pltpu.make_async_copy(sd, dd, rsem).wait()

                    @pl.when(t + KB < nrows)
                    def _():
                        for sd, dd in refill_descs(t + KB):
                            pltpu.async_copy(sd, dd, rsem)

                @pl.when(t < nrows)
                def _():
                    pltpu.async_copy(x_hbm.at[idx_s.at[tw]], rows[b], sems[b])
            return carry
        lax.fori_loop(0, nrows // NB, grp, 0)

    @pl.when(c == 0)
    def _():
        run_list(src2, dst2, TAIL2, TBASE)
        plsc.subcore_barrier()
        pltpu.sync_copy(acc.at[pl.ds(s * ZR, ZR)], p2a.at[pl.ds(s * ZR, ZR)])
        zero_acc()
        plsc.subcore_barrier()
        run_list(src1, dst1, ROWS1, 0)
        plsc.subcore_barrier()

        @pl.when(s < NS - 1)
        def _():
            pltpu.sync_copy(acc.at[pl.ds(s * ZR, ZR)],
                            x1p.at[pl.ds(s * ZR, ZR)])

        @pl.when(s == NS - 1)
        def _():
            pltpu.sync_copy(acc.at[pl.ds((NS - 1) * ZR, LASTR)],
                            x1p.at[pl.ds((NS - 1) * ZR, LASTR)])

    @pl.when(c == 1)
    def _():
        run_list(src2, dst2, HEAD2, 0)
        plsc.subcore_barrier()
        pltpu.sync_copy(acc.at[pl.ds(s * ZR, ZR)], p2b.at[pl.ds(s * ZR, ZR)])


def _merge_body(x1_ref, a_ref, b_ref, o_ref):
    o_ref[:, :D] = x1_ref[...]
    o_ref[:, D:] = a_ref[...] + b_ref[...]


@jax.jit
def _h2gcn(x, src1, dst1, src2, dst2):
    zblk = jnp.zeros((128, D), jnp.float32)
    mesh = plsc.VectorSubcoreMesh(core_axis_name="c", subcore_axis_name="s")
    k = functools.partial(
        pl.kernel,
        mesh=mesh,
        out_type=[
            jax.ShapeDtypeStruct((N_NODES, D), jnp.float32),
            jax.ShapeDtypeStruct((ACC_R, D), jnp.float32),
            jax.ShapeDtypeStruct((ACC_R, D), jnp.float32),
        ],
        scratch_types=[
            pltpu.VMEM_SHARED((ACC_R, D), jnp.float32),
            pltpu.VMEM((IW, CH), jnp.int32),
            pltpu.VMEM((IW, CH), jnp.int32),
            pltpu.VMEM((CH, D), jnp.float32),
            pltpu.VMEM((CH, D), jnp.float32),
            pltpu.SemaphoreType.DMA,
            pltpu.SemaphoreType.DMA,
            pltpu.SemaphoreType.DMA,
        ],
    )(_body)
    x1p, p2a, p2b = k(x, src1, dst1, src2, dst2, zblk)

    blk = 1000
    return pl.pallas_call(
        _merge_body,
        grid=(N_NODES // blk,),
        in_specs=[
            pl.BlockSpec((blk, D), lambda i: (i, 0)),
            pl.BlockSpec((blk, D), lambda i: (i, 0)),
            pl.BlockSpec((blk, D), lambda i: (i, 0)),
        ],
        out_specs=pl.BlockSpec((blk, 2 * D), lambda i: (i, 0)),
        out_shape=jax.ShapeDtypeStruct((N_NODES, 2 * D), jnp.float32),
    )(x1p, p2a, p2b)


def kernel(x, edge_index, edge_index2):
    src1 = edge_index[0].astype(jnp.int32).reshape(E1 // CH, CH)
    dst1 = edge_index[1].astype(jnp.int32).reshape(E1 // CH, CH)
    src2 = edge_index2[0].astype(jnp.int32).reshape(E2 // CH, CH)
    dst2 = edge_index2[1].astype(jnp.int32).reshape(E2 // CH, CH)
    return _h2gcn(x, src1, dst1, src2, dst2)
